# Initial kernel scaffold; baseline (speedup 1.0000x reference)
import jax
import jax.numpy as jnp
from jax.experimental import pallas as pl


def kernel(features, edges, params):
    raise NotImplementedError("write your pallas kernel here")


# speedup vs baseline: 14.9150x; 14.9150x over previous
import functools
import jax
import jax.numpy as jnp
from jax import lax
from jax.experimental import pallas as pl
from jax.experimental.pallas import tpu as pltpu
from jax.experimental.pallas import tpu_sc as plsc

N = 2048
L = 4
E = 65536
EMB = 256
NHEAD = 4
DH = EMB // NHEAD
DFF = 2048
HID = 512
LH = L * NHEAD

NW = 32
NPASS = 2
ROWS_PER = N // NW // NPASS
ECH = 2048

def _build_counts_body(rows_hbm, cols_hbm, m_hbm, macc, rv, cv):
    wid = lax.axis_index("s") * 2 + lax.axis_index("c")
    zeros16 = jnp.zeros((16,), jnp.float32)
    ones16 = jnp.ones((16,), jnp.float32)
    ngrp = N // 16

    for p in range(NPASS):
        base = (p * NW + wid) * ROWS_PER

        def zero_body(i, _):
            r = i // ngrp
            c = lax.rem(i, ngrp)
            macc[r, pl.ds(c * 16, 16)] = zeros16
            return 0

        lax.fori_loop(0, (ROWS_PER + 1) * ngrp, zero_body, 0, unroll=8)

        def chunk_body(ch, _):
            off = ch * ECH
            pltpu.sync_copy(rows_hbm.at[pl.ds(off, ECH)], rv)
            pltpu.sync_copy(cols_hbm.at[pl.ds(off, ECH)], cv)

            def grp(j, _):
                r16 = rv[pl.ds(j * 16, 16)]
                c16 = cv[pl.ds(j * 16, 16)]
                inb = (c16 >= base) & (c16 < base + ROWS_PER)
                rr = jnp.where(inb, c16 - base, ROWS_PER)
                plsc.addupdate_scatter(macc, [rr, r16], ones16)
                return 0

            lax.fori_loop(0, ECH // 16, grp, 0, unroll=4)
            return 0

        lax.fori_loop(0, E // ECH, chunk_body, 0)
        pltpu.sync_copy(macc.at[pl.ds(0, ROWS_PER)],
                        m_hbm.at[pl.ds(base, ROWS_PER)])


@functools.cache
def _counts_kernel():
    mesh = plsc.VectorSubcoreMesh(core_axis_name="c", subcore_axis_name="s")
    return pl.kernel(
        _build_counts_body,
        mesh=mesh,
        compiler_params=pltpu.CompilerParams(needs_layout_passes=False),
        out_type=jax.ShapeDtypeStruct((N, N), jnp.float32),
        scratch_types=[
            pltpu.VMEM((ROWS_PER + 1, N), jnp.float32),
            pltpu.VMEM((ECH,), jnp.int32),
            pltpu.VMEM((ECH,), jnp.int32),
        ],
    )


def _build_counts(rows, cols):
    return _counts_kernel()(rows, cols)


BR = 256
BS_ATT = 512
BN = 256
TB = BN * L


def _dinv_body(m_ref, o_ref):
    deg = jnp.sum(m_ref[...], axis=1, keepdims=True) + 1.0
    o_ref[...] = lax.rsqrt(deg)


def _attn_body(c_ref, xt_ref, o_ref):
    c = c_ref[0]
    l = pl.program_id(0) // NHEAD
    xt = xt_ref[pl.ds(l, 1), :]
    xmax = jnp.max(xt)
    xmin = jnp.min(xt)
    rowmax = jnp.maximum(c * xmax, c * xmin)
    e = jnp.exp(c * xt - rowmax)
    denom = jnp.sum(e, axis=1, keepdims=True)
    num = jnp.sum(e * xt, axis=1, keepdims=True)
    o_ref[0] = num / denom


def _post_body(x_ref, w_ref, c6_ref, l1w_ref, l1b_ref, p_ref, d1_ref,
               w2_ref, b2_ref, l2w_ref, l2b_ref, dinv_ref, w1_ref, o_ref):
    t6 = jnp.concatenate(
        [x_ref[...], w_ref[...], jnp.ones((TB, 1), jnp.float32)], axis=1)
    yc = jnp.dot(t6, c6_ref[...], preferred_element_type=jnp.float32)
    var = jnp.mean(yc * yc, axis=1, keepdims=True)
    s = lax.rsqrt(var + 1e-5)
    x1 = s * yc * l1w_ref[...] + l1b_ref[...]
    hp = s * jnp.dot(t6, p_ref[...], preferred_element_type=jnp.float32) \
        + d1_ref[...]
    hr = jnp.maximum(hp, 0.0)
    ff = jnp.dot(hr, w2_ref[...], preferred_element_type=jnp.float32) \
        + b2_ref[...]
    z = x1 + ff
    m2 = jnp.mean(z, axis=1, keepdims=True)
    zc = z - m2
    v2 = jnp.mean(zc * zc, axis=1, keepdims=True)
    x2 = zc * lax.rsqrt(v2 + 1e-5) * l2w_ref[...] + l2b_ref[...]
    x2r = x2.reshape(BN, L, EMB)
    pooled = (x2r[:, 0] + x2r[:, 1] + x2r[:, 2] + x2r[:, 3]) * (1.0 / L)
    x0 = jnp.maximum(pooled, 0.0)
    o_ref[...] = dinv_ref[...] * jnp.dot(
        x0, w1_ref[...], preferred_element_type=jnp.float32)


def _agg_mid_body(m_ref, zf_ref, zb_ref, dinv_ref, b_ref, wn_ref, o_ref):
    t = jnp.dot(m_ref[...], zf_ref[...], preferred_element_type=jnp.float32) \
        + zb_ref[...]
    y = jnp.maximum(dinv_ref[...] * t + b_ref[...], 0.0)
    o_ref[...] = dinv_ref[...] * jnp.dot(
        y, wn_ref[...], preferred_element_type=jnp.float32)


def _agg_last_body(m_ref, zf_ref, zb_ref, dinv_ref, b_ref, o_ref):
    t = jnp.dot(m_ref[...], zf_ref[...], preferred_element_type=jnp.float32) \
        + zb_ref[...]
    o_ref[...] = dinv_ref[...] * t + b_ref[...]


def _row(v):
    return v[None, :]


def kernel(features, edges, params):
    p = params
    x = features
    rows = edges[0]
    cols = edges[1]

    v_lin = p['lin_w'][0]
    c_src = p['lin_b']
    vq = v_lin @ p['wq']
    cq = c_src @ p['wq'] + p['bq']
    vk = v_lin @ p['wk']
    vv = v_lin @ p['wv']
    cv = c_src @ p['wv'] + p['bv']
    scale = 1.0 / jnp.sqrt(jnp.float32(DH))
    vqh = vq.reshape(NHEAD, DH)
    vkh = vk.reshape(NHEAD, DH)
    cqh = cq.reshape(NHEAD, DH)
    alpha = (vqh * vkh).sum(-1) * scale
    gamma = (cqh * vkh).sum(-1) * scale
    c_arr = (x[:, :, None] * alpha[None, None, :]
             + gamma[None, None, :]).reshape(N, LH)
    xT = x.T

    u = jnp.einsum('hd,hde->he', vv.reshape(NHEAD, DH),
                   p['wo'].reshape(NHEAD, DH, EMB))
    c_sa = cv @ p['wo'] + p['bo']
    B5 = jnp.concatenate([v_lin[None, :], u], axis=0)
    c0 = c_src + c_sa
    C6 = jnp.concatenate([B5 - B5.mean(1, keepdims=True),
                          (c0 - c0.mean())[None, :]], axis=0)
    P = (C6 * p['ln1_w'][None, :]) @ p['ff_w1']
    d1 = _row(p['ln1_b'] @ p['ff_w1'] + p['ff_b1'])

    M = _build_counts(rows, cols)

    dinv = pl.pallas_call(
        _dinv_body,
        grid=(N // BR,),
        in_specs=[pl.BlockSpec((BR, N), lambda i: (i, 0))],
        out_specs=pl.BlockSpec((BR, 1), lambda i: (i, 0)),
        out_shape=jax.ShapeDtypeStruct((N, 1), jnp.float32),
    )(M)

    c3 = c_arr.T[:, :, None]
    W3 = pl.pallas_call(
        _attn_body,
        grid=(LH, N // BS_ATT),
        in_specs=[
            pl.BlockSpec((1, BS_ATT, 1), lambda bh, i: (bh, i, 0)),
            pl.BlockSpec((L, N), lambda bh, i: (0, 0)),
        ],
        out_specs=pl.BlockSpec((1, BS_ATT, 1), lambda bh, i: (bh, i, 0)),
        out_shape=jax.ShapeDtypeStruct((LH, N, 1), jnp.float32),
    )(c3, xT)
    W = W3[:, :, 0].T

    cmap = lambda i: (0, 0)
    Z1 = pl.pallas_call(
        _post_body,
        grid=(N // BN,),
        in_specs=[
            pl.BlockSpec((TB, 1), lambda i: (i, 0)),
            pl.BlockSpec((TB, NHEAD), lambda i: (i, 0)),
            pl.BlockSpec((6, EMB), cmap),
            pl.BlockSpec((1, EMB), cmap),
            pl.BlockSpec((1, EMB), cmap),
            pl.BlockSpec((6, DFF), cmap),
            pl.BlockSpec((1, DFF), cmap),
            pl.BlockSpec((DFF, EMB), cmap),
            pl.BlockSpec((1, EMB), cmap),
            pl.BlockSpec((1, EMB), cmap),
            pl.BlockSpec((1, EMB), cmap),
            pl.BlockSpec((BN, 1), lambda i: (i, 0)),
            pl.BlockSpec((EMB, HID), cmap),
        ],
        out_specs=pl.BlockSpec((BN, HID), lambda i: (i, 0)),
        out_shape=jax.ShapeDtypeStruct((N, HID), jnp.float32),
    )(x.reshape(N * L, 1), W.reshape(N * L, NHEAD), C6,
      _row(p['ln1_w']), _row(p['ln1_b']), P, d1,
      p['ff_w2'], _row(p['ff_b2']), _row(p['ln2_w']), _row(p['ln2_b']),
      dinv, p['conv1_w'])

    def agg_mid(Z, b, wn):
        return pl.pallas_call(
            _agg_mid_body,
            grid=(N // BR,),
            in_specs=[
                pl.BlockSpec((BR, N), lambda i: (i, 0)),
                pl.BlockSpec((N, HID), lambda i: (0, 0)),
                pl.BlockSpec((BR, HID), lambda i: (i, 0)),
                pl.BlockSpec((BR, 1), lambda i: (i, 0)),
                pl.BlockSpec((1, HID), lambda i: (0, 0)),
                pl.BlockSpec((HID, HID), lambda i: (0, 0)),
            ],
            out_specs=pl.BlockSpec((BR, HID), lambda i: (i, 0)),
            out_shape=jax.ShapeDtypeStruct((N, HID), jnp.float32),
        )(M, Z, Z, dinv, _row(b), wn)

    Z2 = agg_mid(Z1, p['conv1_b'], p['conv2_w'])
    Z3 = agg_mid(Z2, p['conv2_b'], p['conv3_w'])
    Z4 = agg_mid(Z3, p['conv3_b'], p['conv4_w'])

    out = pl.pallas_call(
        _agg_last_body,
        grid=(N // BR,),
        in_specs=[
            pl.BlockSpec((BR, N), lambda i: (i, 0)),
            pl.BlockSpec((N, HID), lambda i: (0, 0)),
            pl.BlockSpec((BR, HID), lambda i: (i, 0)),
            pl.BlockSpec((BR, 1), lambda i: (i, 0)),
            pl.BlockSpec((1, HID), lambda i: (0, 0)),
        ],
        out_specs=pl.BlockSpec((BR, HID), lambda i: (i, 0)),
        out_shape=jax.ShapeDtypeStruct((N, HID), jnp.float32),
    )(M, Z4, Z4, dinv, _row(p['conv4_b']))
    return out


# speedup vs baseline: 19.2773x; 1.2925x over previous
import functools
import jax
import jax.numpy as jnp
from jax import lax
from jax.experimental import pallas as pl
from jax.experimental.pallas import tpu as pltpu
from jax.experimental.pallas import tpu_sc as plsc

N = 2048
L = 4
E = 65536
EMB = 256
NHEAD = 4
DH = EMB // NHEAD
DFF = 2048
HID = 512
LH = L * NHEAD

NW = 32
NPASS = 2
ROWS_PER = N // NW // NPASS
ECH = 4096
NCH = E // ECH


def _build_counts_body(rows_hbm, cols_hbm, zer_hbm, m_hbm,
                       macc, rv0, cv0, rv1, cv1,
                       semz, s0r, s0c, s1r, s1c):
    wid = lax.axis_index("s") * 2 + lax.axis_index("c")
    ones16 = jnp.ones((16,), jnp.float32)
    bufs = [(rv0, cv0), (rv1, cv1)]
    sems = [(s0r, s0c), (s1r, s1c)]
    rcap = jnp.uint32(ROWS_PER)

    for p in range(NPASS):
        base = (p * NW + wid) * ROWS_PER
        hz = pltpu.async_copy(zer_hbm, macc, semz)
        hr = [None, None]
        hc = [None, None]
        hr[0] = pltpu.async_copy(rows_hbm.at[pl.ds(0, ECH)], rv0, s0r)
        hc[0] = pltpu.async_copy(cols_hbm.at[pl.ds(0, ECH)], cv0, s0c)
        hz.wait()
        for ch in range(NCH):
            b = ch & 1
            if ch + 1 < NCH:
                nb = (ch + 1) & 1
                off = (ch + 1) * ECH
                hr[nb] = pltpu.async_copy(
                    rows_hbm.at[pl.ds(off, ECH)], bufs[nb][0], sems[nb][0])
                hc[nb] = pltpu.async_copy(
                    cols_hbm.at[pl.ds(off, ECH)], bufs[nb][1], sems[nb][1])
            hr[b].wait()
            hc[b].wait()
            rv, cv = bufs[b]

            def grp(j, _, rv=rv, cv=cv, base=base):
                r16 = rv[pl.ds(j * 16, 16)]
                c16 = cv[pl.ds(j * 16, 16)]
                d = plsc.bitcast(c16 - base, jnp.uint32)
                rr = plsc.bitcast(jnp.minimum(d, rcap), jnp.int32)
                plsc.addupdate_scatter(macc, [rr, r16], ones16)
                return 0

            lax.fori_loop(0, ECH // 16, grp, 0, unroll=8)
        pltpu.sync_copy(macc.at[pl.ds(0, ROWS_PER)],
                        m_hbm.at[pl.ds(base, ROWS_PER)])


@functools.cache
def _counts_kernel():
    mesh = plsc.VectorSubcoreMesh(core_axis_name="c", subcore_axis_name="s")
    return pl.kernel(
        _build_counts_body,
        mesh=mesh,
        compiler_params=pltpu.CompilerParams(needs_layout_passes=False),
        out_type=jax.ShapeDtypeStruct((N, N), jnp.float32),
        scratch_types=[
            pltpu.VMEM((ROWS_PER + 1, N), jnp.float32),
            pltpu.VMEM((ECH,), jnp.int32),
            pltpu.VMEM((ECH,), jnp.int32),
            pltpu.VMEM((ECH,), jnp.int32),
            pltpu.VMEM((ECH,), jnp.int32),
            pltpu.SemaphoreType.DMA,
            pltpu.SemaphoreType.DMA,
            pltpu.SemaphoreType.DMA,
            pltpu.SemaphoreType.DMA,
            pltpu.SemaphoreType.DMA,
        ],
    )


def _build_counts(rows, cols):
    zer = jnp.zeros((ROWS_PER + 1, N), jnp.float32)
    return _counts_kernel()(rows, cols, zer)


BR = 256
BS_ATT = 512
BN = 256
TB = BN * L


def _dinv_body(m_ref, o_ref):
    deg = jnp.sum(m_ref[...], axis=1, keepdims=True) + 1.0
    o_ref[...] = lax.rsqrt(deg)


def _attn_body(c_ref, xt_ref, o_ref):
    c = c_ref[0]
    l = pl.program_id(0) // NHEAD
    xt = xt_ref[pl.ds(l, 1), :]
    xmax = jnp.max(xt)
    xmin = jnp.min(xt)
    rowmax = jnp.maximum(c * xmax, c * xmin)
    e = jnp.exp(c * xt - rowmax)
    denom = jnp.sum(e, axis=1, keepdims=True)
    num = jnp.sum(e * xt, axis=1, keepdims=True)
    o_ref[0] = num / denom


def _post_body(x_ref, w_ref, c6_ref, l1w_ref, l1b_ref, p_ref, d1_ref,
               w2_ref, b2_ref, l2w_ref, l2b_ref, dinv_ref, w1_ref, o_ref):
    t6 = jnp.concatenate(
        [x_ref[...], w_ref[...], jnp.ones((TB, 1), jnp.float32)], axis=1)
    yc = jnp.dot(t6, c6_ref[...], preferred_element_type=jnp.float32)
    var = jnp.mean(yc * yc, axis=1, keepdims=True)
    s = lax.rsqrt(var + 1e-5)
    x1 = s * yc * l1w_ref[...] + l1b_ref[...]
    hp = s * jnp.dot(t6, p_ref[...], preferred_element_type=jnp.float32) \
        + d1_ref[...]
    hr = jnp.maximum(hp, 0.0)
    ff = jnp.dot(hr, w2_ref[...], preferred_element_type=jnp.float32) \
        + b2_ref[...]
    z = x1 + ff
    m2 = jnp.mean(z, axis=1, keepdims=True)
    zc = z - m2
    v2 = jnp.mean(zc * zc, axis=1, keepdims=True)
    x2 = zc * lax.rsqrt(v2 + 1e-5) * l2w_ref[...] + l2b_ref[...]
    x2r = x2.reshape(BN, L, EMB)
    pooled = (x2r[:, 0] + x2r[:, 1] + x2r[:, 2] + x2r[:, 3]) * (1.0 / L)
    x0 = jnp.maximum(pooled, 0.0)
    o_ref[...] = dinv_ref[...] * jnp.dot(
        x0, w1_ref[...], preferred_element_type=jnp.float32)


def _agg_mid_body(m_ref, zf_ref, zb_ref, dinv_ref, b_ref, wn_ref, o_ref):
    t = jnp.dot(m_ref[...], zf_ref[...], preferred_element_type=jnp.float32) \
        + zb_ref[...]
    y = jnp.maximum(dinv_ref[...] * t + b_ref[...], 0.0)
    o_ref[...] = dinv_ref[...] * jnp.dot(
        y, wn_ref[...], preferred_element_type=jnp.float32)


def _agg_last_body(m_ref, zf_ref, zb_ref, dinv_ref, b_ref, o_ref):
    t = jnp.dot(m_ref[...], zf_ref[...], preferred_element_type=jnp.float32) \
        + zb_ref[...]
    o_ref[...] = dinv_ref[...] * t + b_ref[...]


def _row(v):
    return v[None, :]


def kernel(features, edges, params):
    p = params
    x = features
    rows = edges[0]
    cols = edges[1]

    v_lin = p['lin_w'][0]
    c_src = p['lin_b']
    vq = v_lin @ p['wq']
    cq = c_src @ p['wq'] + p['bq']
    vk = v_lin @ p['wk']
    vv = v_lin @ p['wv']
    cv = c_src @ p['wv'] + p['bv']
    scale = 1.0 / jnp.sqrt(jnp.float32(DH))
    vqh = vq.reshape(NHEAD, DH)
    vkh = vk.reshape(NHEAD, DH)
    cqh = cq.reshape(NHEAD, DH)
    alpha = (vqh * vkh).sum(-1) * scale
    gamma = (cqh * vkh).sum(-1) * scale
    c_arr = (x[:, :, None] * alpha[None, None, :]
             + gamma[None, None, :]).reshape(N, LH)
    xT = x.T

    u = jnp.einsum('hd,hde->he', vv.reshape(NHEAD, DH),
                   p['wo'].reshape(NHEAD, DH, EMB))
    c_sa = cv @ p['wo'] + p['bo']
    B5 = jnp.concatenate([v_lin[None, :], u], axis=0)
    c0 = c_src + c_sa
    C6 = jnp.concatenate([B5 - B5.mean(1, keepdims=True),
                          (c0 - c0.mean())[None, :]], axis=0)
    P = (C6 * p['ln1_w'][None, :]) @ p['ff_w1']
    d1 = _row(p['ln1_b'] @ p['ff_w1'] + p['ff_b1'])

    M = _build_counts(rows, cols)

    dinv = pl.pallas_call(
        _dinv_body,
        grid=(N // BR,),
        in_specs=[pl.BlockSpec((BR, N), lambda i: (i, 0))],
        out_specs=pl.BlockSpec((BR, 1), lambda i: (i, 0)),
        out_shape=jax.ShapeDtypeStruct((N, 1), jnp.float32),
    )(M)

    c3 = c_arr.T[:, :, None]
    W3 = pl.pallas_call(
        _attn_body,
        grid=(LH, N // BS_ATT),
        in_specs=[
            pl.BlockSpec((1, BS_ATT, 1), lambda bh, i: (bh, i, 0)),
            pl.BlockSpec((L, N), lambda bh, i: (0, 0)),
        ],
        out_specs=pl.BlockSpec((1, BS_ATT, 1), lambda bh, i: (bh, i, 0)),
        out_shape=jax.ShapeDtypeStruct((LH, N, 1), jnp.float32),
    )(c3, xT)
    W = W3[:, :, 0].T

    cmap = lambda i: (0, 0)
    Z1 = pl.pallas_call(
        _post_body,
        grid=(N // BN,),
        in_specs=[
            pl.BlockSpec((TB, 1), lambda i: (i, 0)),
            pl.BlockSpec((TB, NHEAD), lambda i: (i, 0)),
            pl.BlockSpec((6, EMB), cmap),
            pl.BlockSpec((1, EMB), cmap),
            pl.BlockSpec((1, EMB), cmap),
            pl.BlockSpec((6, DFF), cmap),
            pl.BlockSpec((1, DFF), cmap),
            pl.BlockSpec((DFF, EMB), cmap),
            pl.BlockSpec((1, EMB), cmap),
            pl.BlockSpec((1, EMB), cmap),
            pl.BlockSpec((1, EMB), cmap),
            pl.BlockSpec((BN, 1), lambda i: (i, 0)),
            pl.BlockSpec((EMB, HID), cmap),
        ],
        out_specs=pl.BlockSpec((BN, HID), lambda i: (i, 0)),
        out_shape=jax.ShapeDtypeStruct((N, HID), jnp.float32),
    )(x.reshape(N * L, 1), W.reshape(N * L, NHEAD), C6,
      _row(p['ln1_w']), _row(p['ln1_b']), P, d1,
      p['ff_w2'], _row(p['ff_b2']), _row(p['ln2_w']), _row(p['ln2_b']),
      dinv, p['conv1_w'])

    def agg_mid(Z, b, wn):
        return pl.pallas_call(
            _agg_mid_body,
            grid=(N // BR,),
            in_specs=[
                pl.BlockSpec((BR, N), lambda i: (i, 0)),
                pl.BlockSpec((N, HID), lambda i: (0, 0)),
                pl.BlockSpec((BR, HID), lambda i: (i, 0)),
                pl.BlockSpec((BR, 1), lambda i: (i, 0)),
                pl.BlockSpec((1, HID), lambda i: (0, 0)),
                pl.BlockSpec((HID, HID), lambda i: (0, 0)),
            ],
            out_specs=pl.BlockSpec((BR, HID), lambda i: (i, 0)),
            out_shape=jax.ShapeDtypeStruct((N, HID), jnp.float32),
        )(M, Z, Z, dinv, _row(b), wn)

    Z2 = agg_mid(Z1, p['conv1_b'], p['conv2_w'])
    Z3 = agg_mid(Z2, p['conv2_b'], p['conv3_w'])
    Z4 = agg_mid(Z3, p['conv3_b'], p['conv4_w'])

    out = pl.pallas_call(
        _agg_last_body,
        grid=(N // BR,),
        in_specs=[
            pl.BlockSpec((BR, N), lambda i: (i, 0)),
            pl.BlockSpec((N, HID), lambda i: (0, 0)),
            pl.BlockSpec((BR, HID), lambda i: (i, 0)),
            pl.BlockSpec((BR, 1), lambda i: (i, 0)),
            pl.BlockSpec((1, HID), lambda i: (0, 0)),
        ],
        out_specs=pl.BlockSpec((BR, HID), lambda i: (i, 0)),
        out_shape=jax.ShapeDtypeStruct((N, HID), jnp.float32),
    )(M, Z4, Z4, dinv, _row(p['conv4_b']))
    return out
